# initial kernel scaffold (unmeasured)
import jax
import jax.numpy as jnp
from jax import lax
from jax.experimental import pallas as pl
from jax.experimental.pallas import tpu as pltpu

T = 1024
D = 2048
V_HALF = 16384
V = 2 * V_HALF
TILE = 512
N_TILES = V_HALF // TILE
SLANES = 128


def _gemm_stats(x, W):

    def body(x_ref, w_ref, logits_ref, m_ref, s_ref):
        j = pl.program_id(0)
        logits = jnp.dot(x_ref[...], w_ref[...],
                         preferred_element_type=jnp.float32)
        logits_ref[...] = logits
        m_t = jnp.max(logits, axis=1, keepdims=True)
        s_t = jnp.sum(jnp.exp(logits - m_t), axis=1, keepdims=True)

        @pl.when(j == 0)
        def _():
            m_ref[...] = jnp.broadcast_to(m_t, (T, SLANES))
            s_ref[...] = jnp.broadcast_to(s_t, (T, SLANES))

        @pl.when(j > 0)
        def _():
            m_old = m_ref[:, :1]
            s_old = s_ref[:, :1]
            m_new = jnp.maximum(m_old, m_t)
            s_new = (s_old * jnp.exp(m_old - m_new)
                     + s_t * jnp.exp(m_t - m_new))
            m_ref[...] = jnp.broadcast_to(m_new, (T, SLANES))
            s_ref[...] = jnp.broadcast_to(s_new, (T, SLANES))

    return pl.pallas_call(
        body,
        grid=(N_TILES,),
        in_specs=[
            pl.BlockSpec((T, D), lambda j: (0, 0)),
            pl.BlockSpec((D, TILE), lambda j: (0, j)),
        ],
        out_specs=[
            pl.BlockSpec((T, TILE), lambda j: (0, j)),
            pl.BlockSpec((T, SLANES), lambda j: (0, 0)),
            pl.BlockSpec((T, SLANES), lambda j: (0, 0)),
        ],
        out_shape=[
            jax.ShapeDtypeStruct((T, V_HALF), jnp.float32),
            jax.ShapeDtypeStruct((T, SLANES), jnp.float32),
            jax.ShapeDtypeStruct((T, SLANES), jnp.float32),
        ],
    )(x, W)


def _stats_allreduce(m, s):

    def body(m_ref, s_ref, c_ref, send_buf, recv_buf, send_sem, recv_sem):
        my_x = lax.axis_index("x")
        my_y = lax.axis_index("y")
        send_buf[0, :, :] = m_ref[...]
        send_buf[1, :, :] = s_ref[...]
        rdma = pltpu.make_async_remote_copy(
            src_ref=send_buf,
            dst_ref=recv_buf,
            send_sem=send_sem,
            recv_sem=recv_sem,
            device_id=(my_x, 1 - my_y),
            device_id_type=pl.DeviceIdType.MESH,
        )
        rdma.start()
        rdma.wait()
        m_l = m_ref[:, :1]
        s_l = s_ref[:, :1]
        m_o = recv_buf[0, :, :1]
        s_o = recv_buf[1, :, :1]
        m_g = jnp.maximum(m_l, m_o)
        z_g = s_l * jnp.exp(m_l - m_g) + s_o * jnp.exp(m_o - m_g)
        c_ref[...] = jnp.broadcast_to(m_g + jnp.log(z_g), (T, SLANES))

    return pl.pallas_call(
        body,
        in_specs=[
            pl.BlockSpec(memory_space=pltpu.VMEM),
            pl.BlockSpec(memory_space=pltpu.VMEM),
        ],
        out_specs=pl.BlockSpec(memory_space=pltpu.VMEM),
        out_shape=jax.ShapeDtypeStruct((T, SLANES), jnp.float32),
        scratch_shapes=[
            pltpu.VMEM((2, T, SLANES), jnp.float32),
            pltpu.VMEM((2, T, SLANES), jnp.float32),
            pltpu.SemaphoreType.DMA,
            pltpu.SemaphoreType.DMA,
        ],
        compiler_params=pltpu.CompilerParams(has_side_effects=True),
    )(m, s)


def _normalize_exchange(logits, c):

    def body(logits_ref, c_ref, out_ref, tile, copy_sem, send_sems, recv_sems):
        j = pl.program_id(0)
        my_x = lax.axis_index("x")
        my_y = lax.axis_index("y")
        nbr = (my_x, 1 - my_y)

        tile[...] = jnp.exp(logits_ref[...] - c_ref[:, :1])
        col = my_y * V_HALF + j * TILE

        cp = pltpu.make_async_copy(
            tile, out_ref.at[:, pl.ds(col, TILE)], copy_sem)
        cp.start()
        rdma = pltpu.make_async_remote_copy(
            src_ref=tile,
            dst_ref=out_ref.at[:, pl.ds(col, TILE)],
            send_sem=send_sems.at[j],
            recv_sem=recv_sems.at[j],
            device_id=nbr,
            device_id_type=pl.DeviceIdType.MESH,
        )
        rdma.start()
        cp.wait()
        rdma.wait_send()

        @pl.when(j == N_TILES - 1)
        def _():
            nbr_base = (1 - my_y) * V_HALF
            for i in range(N_TILES):
                r = pltpu.make_async_remote_copy(
                    src_ref=tile,
                    dst_ref=out_ref.at[:, pl.ds(nbr_base + i * TILE, TILE)],
                    send_sem=send_sems.at[i],
                    recv_sem=recv_sems.at[i],
                    device_id=nbr,
                    device_id_type=pl.DeviceIdType.MESH,
                )
                r.wait_recv()

    return pl.pallas_call(
        body,
        grid=(N_TILES,),
        in_specs=[
            pl.BlockSpec((T, TILE), lambda j: (0, j)),
            pl.BlockSpec((T, SLANES), lambda j: (0, 0)),
        ],
        out_specs=pl.BlockSpec(memory_space=pltpu.ANY),
        out_shape=jax.ShapeDtypeStruct((T, V), jnp.float32),
        scratch_shapes=[
            pltpu.VMEM((T, TILE), jnp.float32),
            pltpu.SemaphoreType.DMA,
            pltpu.SemaphoreType.DMA((N_TILES,)),
            pltpu.SemaphoreType.DMA((N_TILES,)),
        ],
        compiler_params=pltpu.CompilerParams(has_side_effects=True),
    )(logits, c)


def kernel(x, W):
    logits, m, s = _gemm_stats(x, W)
    c = _stats_allreduce(m, s)
    return _normalize_exchange(logits, c)


# baseline (device time: 964917 ns/iter reference)
import jax
import jax.numpy as jnp
from jax import lax
from jax.experimental import pallas as pl
from jax.experimental.pallas import tpu as pltpu

T = 1024
D = 2048
V_HALF = 16384
V = 2 * V_HALF
TILE = 512
N_TILES = V_HALF // TILE
SLANES = 128


def _gemm_stats(x, W):

    def body(x_ref, w_ref, logits_ref, m_ref, s_ref):
        j = pl.program_id(0)
        logits = jnp.dot(x_ref[...], w_ref[...],
                         preferred_element_type=jnp.float32)
        logits_ref[...] = logits
        m_t = jnp.max(logits, axis=1, keepdims=True)
        s_t = jnp.sum(jnp.exp(logits - m_t), axis=1, keepdims=True)

        @pl.when(j == 0)
        def _():
            m_ref[...] = jnp.broadcast_to(m_t, (T, SLANES))
            s_ref[...] = jnp.broadcast_to(s_t, (T, SLANES))

        @pl.when(j > 0)
        def _():
            m_old = m_ref[:, :1]
            s_old = s_ref[:, :1]
            m_new = jnp.maximum(m_old, m_t)
            s_new = (s_old * jnp.exp(m_old - m_new)
                     + s_t * jnp.exp(m_t - m_new))
            m_ref[...] = jnp.broadcast_to(m_new, (T, SLANES))
            s_ref[...] = jnp.broadcast_to(s_new, (T, SLANES))

    return pl.pallas_call(
        body,
        grid=(N_TILES,),
        in_specs=[
            pl.BlockSpec((T, D), lambda j: (0, 0)),
            pl.BlockSpec((D, TILE), lambda j: (0, j)),
        ],
        out_specs=[
            pl.BlockSpec((T, TILE), lambda j: (0, j)),
            pl.BlockSpec((T, SLANES), lambda j: (0, 0)),
            pl.BlockSpec((T, SLANES), lambda j: (0, 0)),
        ],
        out_shape=[
            jax.ShapeDtypeStruct((T, V_HALF), jnp.float32),
            jax.ShapeDtypeStruct((T, SLANES), jnp.float32),
            jax.ShapeDtypeStruct((T, SLANES), jnp.float32),
        ],
    )(x, W)


def _stats_allreduce(m, s):

    def body(m_ref, s_ref, c_ref, send_buf, recv_buf, send_sem, recv_sem):
        my_x = lax.axis_index("x")
        my_y = lax.axis_index("y")
        send_buf[0, :, :] = m_ref[...]
        send_buf[1, :, :] = s_ref[...]
        rdma = pltpu.make_async_remote_copy(
            src_ref=send_buf,
            dst_ref=recv_buf,
            send_sem=send_sem,
            recv_sem=recv_sem,
            device_id=(my_x, 1 - my_y),
            device_id_type=pl.DeviceIdType.MESH,
        )
        rdma.start()
        rdma.wait()
        m_l = m_ref[:, :1]
        s_l = s_ref[:, :1]
        m_o = recv_buf[0, :, :1]
        s_o = recv_buf[1, :, :1]
        m_g = jnp.maximum(m_l, m_o)
        z_g = s_l * jnp.exp(m_l - m_g) + s_o * jnp.exp(m_o - m_g)
        c_ref[...] = jnp.broadcast_to(m_g + jnp.log(z_g), (T, SLANES))

    return pl.pallas_call(
        body,
        in_specs=[
            pl.BlockSpec(memory_space=pltpu.VMEM),
            pl.BlockSpec(memory_space=pltpu.VMEM),
        ],
        out_specs=pl.BlockSpec(memory_space=pltpu.VMEM),
        out_shape=jax.ShapeDtypeStruct((T, SLANES), jnp.float32),
        scratch_shapes=[
            pltpu.VMEM((2, T, SLANES), jnp.float32),
            pltpu.VMEM((2, T, SLANES), jnp.float32),
            pltpu.SemaphoreType.DMA,
            pltpu.SemaphoreType.DMA,
        ],
        compiler_params=pltpu.CompilerParams(has_side_effects=True),
    )(m, s)


def _normalize_exchange(logits, c):

    def body(logits_ref, c_ref, out_ref, tile, copy_sem, send_sems, recv_sems):
        j = pl.program_id(0)
        my_x = lax.axis_index("x")
        my_y = lax.axis_index("y")
        nbr = (my_x, 1 - my_y)

        tile[...] = jnp.exp(logits_ref[...] - c_ref[:, :1])
        col = my_y * V_HALF + j * TILE

        cp = pltpu.make_async_copy(
            tile, out_ref.at[:, pl.ds(col, TILE)], copy_sem)
        cp.start()
        rdma = pltpu.make_async_remote_copy(
            src_ref=tile,
            dst_ref=out_ref.at[:, pl.ds(col, TILE)],
            send_sem=send_sems.at[j],
            recv_sem=recv_sems.at[j],
            device_id=nbr,
            device_id_type=pl.DeviceIdType.MESH,
        )
        rdma.start()
        cp.wait()
        rdma.wait_send()

        @pl.when(j == N_TILES - 1)
        def _():
            nbr_base = (1 - my_y) * V_HALF
            for i in range(N_TILES):
                r = pltpu.make_async_remote_copy(
                    src_ref=tile,
                    dst_ref=out_ref.at[:, pl.ds(nbr_base + i * TILE, TILE)],
                    send_sem=send_sems.at[i],
                    recv_sem=recv_sems.at[i],
                    device_id=nbr,
                    device_id_type=pl.DeviceIdType.MESH,
                )
                r.wait_recv()

    return pl.pallas_call(
        body,
        grid=(N_TILES,),
        in_specs=[
            pl.BlockSpec((T, TILE), lambda j: (0, j)),
            pl.BlockSpec((T, SLANES), lambda j: (0, 0)),
        ],
        out_specs=pl.BlockSpec(memory_space=pl.ANY),
        out_shape=jax.ShapeDtypeStruct((T, V), jnp.float32),
        scratch_shapes=[
            pltpu.VMEM((T, TILE), jnp.float32),
            pltpu.SemaphoreType.DMA,
            pltpu.SemaphoreType.DMA((N_TILES,)),
            pltpu.SemaphoreType.DMA((N_TILES,)),
        ],
        compiler_params=pltpu.CompilerParams(has_side_effects=True),
    )(logits, c)


def kernel(x, W):
    logits, m, s = _gemm_stats(x, W)
    c = _stats_allreduce(m, s)
    return _normalize_exchange(logits, c)


# device time: 867917 ns/iter; 1.1118x vs baseline; 1.1118x over previous
import jax
import jax.numpy as jnp
from jax import lax
from jax.experimental import pallas as pl
from jax.experimental.pallas import tpu as pltpu

T = 1024
D = 2048
V_HALF = 16384
V = 2 * V_HALF
TILE = 512
N_TILES = V_HALF // TILE
K = 9
SLANES = 128


def _gemm_headsend(x, W):

    def body(x_ref, w_ref, logits_ref, c_ref, nbr_raw_ref,
             m_ref, s_ref, head, stat_buf, stat_recv,
             head_send_sems, head_recv_sems, stat_send_sem, stat_recv_sem):
        j = pl.program_id(0)
        my_x = lax.axis_index("x")
        my_y = lax.axis_index("y")
        nbr = (my_x, 1 - my_y)

        logits = jnp.dot(x_ref[...], w_ref[...],
                         preferred_element_type=jnp.float32)
        logits_ref[...] = logits
        m_t = jnp.max(logits, axis=1, keepdims=True)
        s_t = jnp.sum(jnp.exp(logits - m_t), axis=1, keepdims=True)

        @pl.when(j == 0)
        def _():
            m_ref[...] = jnp.broadcast_to(m_t, (T, SLANES))
            s_ref[...] = jnp.broadcast_to(s_t, (T, SLANES))

        @pl.when(j > 0)
        def _():
            m_old = m_ref[:, :1]
            s_old = s_ref[:, :1]
            m_new = jnp.maximum(m_old, m_t)
            s_new = (s_old * jnp.exp(m_old - m_new)
                     + s_t * jnp.exp(m_t - m_new))
            m_ref[...] = jnp.broadcast_to(m_new, (T, SLANES))
            s_ref[...] = jnp.broadcast_to(s_new, (T, SLANES))

        def head_rdma(i):
            return pltpu.make_async_remote_copy(
                src_ref=head.at[i],
                dst_ref=nbr_raw_ref.at[:, pl.ds(i * TILE, TILE)],
                send_sem=head_send_sems.at[i],
                recv_sem=head_recv_sems.at[i],
                device_id=nbr,
                device_id_type=pl.DeviceIdType.MESH,
            )

        for i in range(K):
            @pl.when(j == i)
            def _(i=i):
                head[i, :, :] = logits
                head_rdma(i).start()

        @pl.when(j == N_TILES - 1)
        def _():
            for i in range(K):
                head_rdma(i).wait()
            stat_buf[0, :, :] = m_ref[...]
            stat_buf[1, :, :] = s_ref[...]
            rs = pltpu.make_async_remote_copy(
                src_ref=stat_buf,
                dst_ref=stat_recv,
                send_sem=stat_send_sem,
                recv_sem=stat_recv_sem,
                device_id=nbr,
                device_id_type=pl.DeviceIdType.MESH,
            )
            rs.start()
            rs.wait()
            m_l = m_ref[:, :1]
            s_l = s_ref[:, :1]
            m_o = stat_recv[0, :, :1]
            s_o = stat_recv[1, :, :1]
            m_g = jnp.maximum(m_l, m_o)
            z_g = s_l * jnp.exp(m_l - m_g) + s_o * jnp.exp(m_o - m_g)
            c_ref[...] = jnp.broadcast_to(m_g + jnp.log(z_g), (T, SLANES))

    return pl.pallas_call(
        body,
        grid=(N_TILES,),
        in_specs=[
            pl.BlockSpec((T, D), lambda j: (0, 0)),
            pl.BlockSpec((D, TILE), lambda j: (0, j)),
        ],
        out_specs=[
            pl.BlockSpec((T, TILE), lambda j: (0, j)),
            pl.BlockSpec((T, SLANES), lambda j: (0, 0)),
            pl.BlockSpec(memory_space=pl.ANY),
        ],
        out_shape=[
            jax.ShapeDtypeStruct((T, V_HALF), jnp.float32),
            jax.ShapeDtypeStruct((T, SLANES), jnp.float32),
            jax.ShapeDtypeStruct((T, K * TILE), jnp.float32),
        ],
        scratch_shapes=[
            pltpu.VMEM((T, SLANES), jnp.float32),
            pltpu.VMEM((T, SLANES), jnp.float32),
            pltpu.VMEM((K, T, TILE), jnp.float32),
            pltpu.VMEM((2, T, SLANES), jnp.float32),
            pltpu.VMEM((2, T, SLANES), jnp.float32),
            pltpu.SemaphoreType.DMA((K,)),
            pltpu.SemaphoreType.DMA((K,)),
            pltpu.SemaphoreType.DMA,
            pltpu.SemaphoreType.DMA,
        ],
        compiler_params=pltpu.CompilerParams(
            has_side_effects=True, vmem_limit_bytes=100 * 1024 * 1024),
    )(x, W)


def _normalize_exchange(logits, c, nbr_raw):

    N_SEND = N_TILES - K

    def body(logits_ref, c_ref, nbr_raw_ref, out_ref,
             snd, raw_t, cp_sems, send_sems, recv_sems, raw_ld_sem,
             raw_st_sem):
        j = pl.program_id(0)
        jt = (j + K) % N_TILES
        slot = lax.rem(j, 2)
        my_x = lax.axis_index("x")
        my_y = lax.axis_index("y")
        nbr = (my_x, 1 - my_y)
        my_col = my_y * V_HALF + jt * TILE
        nbr_col = (1 - my_y) * V_HALF + jt * TILE

        def local_cp(sl, col):
            return pltpu.make_async_copy(
                snd.at[sl], out_ref.at[:, pl.ds(col, TILE)], cp_sems.at[sl])

        def send_rdma(sl, col, tile_idx):
            return pltpu.make_async_remote_copy(
                src_ref=snd.at[sl],
                dst_ref=out_ref.at[:, pl.ds(col, TILE)],
                send_sem=send_sems.at[sl],
                recv_sem=recv_sems.at[tile_idx],
                device_id=nbr,
                device_id_type=pl.DeviceIdType.MESH,
            )

        @pl.when(j >= 2)
        def _():
            jt_prev = (j - 2 + K) % N_TILES
            col_prev = my_y * V_HALF + jt_prev * TILE
            local_cp(slot, col_prev).wait()

            @pl.when(j - 2 < N_SEND)
            def _():
                send_rdma(slot, col_prev, jt_prev).wait_send()

        snd[slot, :, :] = jnp.exp(logits_ref[...] - c_ref[:, :1])
        local_cp(slot, my_col).start()

        @pl.when(j < N_SEND)
        def _():
            send_rdma(slot, my_col, jt).start()

        @pl.when(j >= N_SEND)
        def _():
            ld = pltpu.make_async_copy(
                nbr_raw_ref.at[:, pl.ds(jt * TILE, TILE)], raw_t, raw_ld_sem)
            ld.start()
            ld.wait()
            raw_t[...] = jnp.exp(raw_t[...] - c_ref[:, :1])
            st = pltpu.make_async_copy(
                raw_t, out_ref.at[:, pl.ds(nbr_col, TILE)], raw_st_sem)
            st.start()
            st.wait()

        @pl.when(j == N_TILES - 1)
        def _():
            for dj in (N_TILES - 2, N_TILES - 1):
                sl = dj % 2
                jtp = (dj + K) % N_TILES
                local_cp(sl, my_y * V_HALF + jtp * TILE).wait()
            for i in range(K, N_TILES):
                r = pltpu.make_async_remote_copy(
                    src_ref=snd.at[0],
                    dst_ref=out_ref.at[
                        :, pl.ds((1 - my_y) * V_HALF + i * TILE, TILE)],
                    send_sem=send_sems.at[0],
                    recv_sem=recv_sems.at[i],
                    device_id=nbr,
                    device_id_type=pl.DeviceIdType.MESH,
                )
                r.wait_recv()

    return pl.pallas_call(
        body,
        grid=(N_TILES,),
        in_specs=[
            pl.BlockSpec((T, TILE), lambda j: (0, (j + K) % N_TILES)),
            pl.BlockSpec((T, SLANES), lambda j: (0, 0)),
            pl.BlockSpec(memory_space=pl.ANY),
        ],
        out_specs=pl.BlockSpec(memory_space=pl.ANY),
        out_shape=jax.ShapeDtypeStruct((T, V), jnp.float32),
        scratch_shapes=[
            pltpu.VMEM((2, T, TILE), jnp.float32),
            pltpu.VMEM((T, TILE), jnp.float32),
            pltpu.SemaphoreType.DMA((2,)),
            pltpu.SemaphoreType.DMA((2,)),
            pltpu.SemaphoreType.DMA((N_TILES,)),
            pltpu.SemaphoreType.DMA,
            pltpu.SemaphoreType.DMA,
        ],
        compiler_params=pltpu.CompilerParams(has_side_effects=True),
    )(logits, c, nbr_raw)


def kernel(x, W):
    logits, c, nbr_raw = _gemm_headsend(x, W)
    return _normalize_exchange(logits, c, nbr_raw)
